# Initial kernel scaffold; baseline (speedup 1.0000x reference)
import jax
import jax.numpy as jnp
from jax.experimental import pallas as pl


def kernel(z, embed_weight):
    raise NotImplementedError("write your pallas kernel here")


# speedup vs baseline: 1.3581x; 1.3581x over previous
import jax
import jax.numpy as jnp
from jax.experimental import pallas as pl

_NUM_EMBED = 1024
_EMBED_DIM = 64
_BLK_W = 1024


def _vq_body(e_ref, z_ref, out_ref, sse_ref):
    b = pl.program_id(0)
    w = pl.program_id(1)

    zb = z_ref[0]
    e = e_ref[...]

    eq = jnp.sum(e * e, axis=1, keepdims=True)
    zq = jnp.sum(zb * zb, axis=0, keepdims=True)

    mm = jax.lax.dot_general(
        e, zb, (((1,), (0,)), ((), ())),
        preferred_element_type=jnp.float32)
    dist = (zq + eq) - 2.0 * mm

    m = jnp.min(dist, axis=0, keepdims=True)
    iota = jax.lax.broadcasted_iota(jnp.int32, (_NUM_EMBED, _BLK_W), 0)
    idx = jnp.min(jnp.where(dist == m, iota, _NUM_EMBED), axis=0)
    onehot = (iota == idx[None, :]).astype(jnp.float32)

    q = jax.lax.dot_general(
        e, onehot, (((0,), (0,)), ((), ())),
        precision=jax.lax.Precision.HIGHEST,
        preferred_element_type=jnp.float32)

    out_ref[0] = zb + (q - zb)

    d = zb - q
    part = jnp.sum(d * d).reshape(1, 1)

    @pl.when((b == 0) & (w == 0))
    def _init():
        sse_ref[...] = jnp.zeros((1, 1), jnp.float32)

    sse_ref[...] += part


def kernel(z, embed_weight):
    batch, ch, hh, ww = z.shape
    hw = hh * ww
    zr = z.reshape(batch, ch, hw)

    grid = (batch, hw // _BLK_W)
    out, sse = pl.pallas_call(
        _vq_body,
        grid=grid,
        in_specs=[
            pl.BlockSpec((_NUM_EMBED, _EMBED_DIM), lambda b, w: (0, 0)),
            pl.BlockSpec((1, ch, _BLK_W), lambda b, w: (b, 0, w)),
        ],
        out_specs=[
            pl.BlockSpec((1, ch, _BLK_W), lambda b, w: (b, 0, w)),
            pl.BlockSpec((1, 1), lambda b, w: (0, 0)),
        ],
        out_shape=[
            jax.ShapeDtypeStruct((batch, ch, hw), jnp.float32),
            jax.ShapeDtypeStruct((1, 1), jnp.float32),
        ],
    )(embed_weight, zr)

    quantized_st = out.reshape(batch, ch, hh, ww)
    m = sse[0, 0] / z.size
    loss = 0.25 * m + m
    return quantized_st, loss


# speedup vs baseline: 1.8151x; 1.3364x over previous
import jax
import jax.numpy as jnp
from jax.experimental import pallas as pl

_NUM_EMBED = 1024
_EMBED_DIM = 64
_BLK_W = 1024


def _vq_body(e_ref, z_ref, out_ref, sse_ref):
    b = pl.program_id(0)
    w = pl.program_id(1)

    zb = z_ref[0]
    e = e_ref[...]

    eq = jnp.sum(e * e, axis=1, keepdims=True)
    zq = jnp.sum(zb * zb, axis=0, keepdims=True)

    mm2 = jax.lax.dot_general(
        e + e, zb, (((1,), (0,)), ((), ())),
        preferred_element_type=jnp.float32)
    dist = (zq + eq) - mm2

    m = jnp.min(dist, axis=0, keepdims=True)
    iota = jax.lax.broadcasted_iota(jnp.int32, (_NUM_EMBED, _BLK_W), 0)
    sel = jnp.where(dist == m, _NUM_EMBED - iota, 0)
    mx = jnp.max(sel, axis=0, keepdims=True)
    onehot = (sel == mx).astype(jnp.bfloat16)

    e_hi = e.astype(jnp.bfloat16)
    r1 = e - e_hi.astype(jnp.float32)
    e_mid = r1.astype(jnp.bfloat16)
    e_lo = (r1 - e_mid.astype(jnp.float32)).astype(jnp.bfloat16)

    def _gpass(part):
        return jax.lax.dot_general(
            part, onehot, (((0,), (0,)), ((), ())),
            preferred_element_type=jnp.float32)

    q = (_gpass(e_hi) + _gpass(e_mid)) + _gpass(e_lo)

    out_ref[0] = zb + (q - zb)

    d = zb - q
    part = jnp.sum(d * d).reshape(1, 1)

    @pl.when((b == 0) & (w == 0))
    def _init():
        sse_ref[...] = jnp.zeros((1, 1), jnp.float32)

    sse_ref[...] += part


def kernel(z, embed_weight):
    batch, ch, hh, ww = z.shape
    hw = hh * ww
    zr = z.reshape(batch, ch, hw)

    grid = (batch, hw // _BLK_W)
    out, sse = pl.pallas_call(
        _vq_body,
        grid=grid,
        in_specs=[
            pl.BlockSpec((_NUM_EMBED, _EMBED_DIM), lambda b, w: (0, 0)),
            pl.BlockSpec((1, ch, _BLK_W), lambda b, w: (b, 0, w)),
        ],
        out_specs=[
            pl.BlockSpec((1, ch, _BLK_W), lambda b, w: (b, 0, w)),
            pl.BlockSpec((1, 1), lambda b, w: (0, 0)),
        ],
        out_shape=[
            jax.ShapeDtypeStruct((batch, ch, hw), jnp.float32),
            jax.ShapeDtypeStruct((1, 1), jnp.float32),
        ],
    )(embed_weight, zr)

    quantized_st = out.reshape(batch, ch, hh, ww)
    m = sse[0, 0] / z.size
    loss = 0.25 * m + m
    return quantized_st, loss


# speedup vs baseline: 1.9148x; 1.0549x over previous
import jax
import jax.numpy as jnp
from jax.experimental import pallas as pl

_NUM_EMBED = 1024
_EMBED_DIM = 64
_BLK_W = 2048


def _vq_body(e_ref, z_ref, out_ref, sse_ref):
    b = pl.program_id(0)
    w = pl.program_id(1)

    zb = z_ref[0]
    e = e_ref[...]

    eq = jnp.sum(e * e, axis=1, keepdims=True)
    zq = jnp.sum(zb * zb, axis=0, keepdims=True)

    mm2 = jax.lax.dot_general(
        e + e, zb, (((1,), (0,)), ((), ())),
        preferred_element_type=jnp.float32)
    dist = (zq + eq) - mm2

    m = jnp.min(dist, axis=0, keepdims=True)
    iota = jax.lax.broadcasted_iota(jnp.int32, (_NUM_EMBED, _BLK_W), 0)
    sel = jnp.where(dist == m, _NUM_EMBED - iota, 0)
    mx = jnp.max(sel, axis=0, keepdims=True)
    onehot = (sel == mx).astype(jnp.bfloat16)

    e_hi = e.astype(jnp.bfloat16)
    r1 = e - e_hi.astype(jnp.float32)
    e_mid = r1.astype(jnp.bfloat16)
    e_lo = (r1 - e_mid.astype(jnp.float32)).astype(jnp.bfloat16)

    def _gpass(part):
        return jax.lax.dot_general(
            part, onehot, (((0,), (0,)), ((), ())),
            preferred_element_type=jnp.float32)

    q = (_gpass(e_hi) + _gpass(e_mid)) + _gpass(e_lo)

    out_ref[0] = zb + (q - zb)

    d = zb - q
    part = jnp.sum(d * d).reshape(1, 1)

    @pl.when((b == 0) & (w == 0))
    def _init():
        sse_ref[...] = jnp.zeros((1, 1), jnp.float32)

    sse_ref[...] += part


def kernel(z, embed_weight):
    batch, ch, hh, ww = z.shape
    hw = hh * ww
    zr = z.reshape(batch, ch, hw)

    grid = (batch, hw // _BLK_W)
    out, sse = pl.pallas_call(
        _vq_body,
        grid=grid,
        in_specs=[
            pl.BlockSpec((_NUM_EMBED, _EMBED_DIM), lambda b, w: (0, 0)),
            pl.BlockSpec((1, ch, _BLK_W), lambda b, w: (b, 0, w)),
        ],
        out_specs=[
            pl.BlockSpec((1, ch, _BLK_W), lambda b, w: (b, 0, w)),
            pl.BlockSpec((1, 1), lambda b, w: (0, 0)),
        ],
        out_shape=[
            jax.ShapeDtypeStruct((batch, ch, hw), jnp.float32),
            jax.ShapeDtypeStruct((1, 1), jnp.float32),
        ],
    )(embed_weight, zr)

    quantized_st = out.reshape(batch, ch, hh, ww)
    m = sse[0, 0] / z.size
    loss = 0.25 * m + m
    return quantized_st, loss


# speedup vs baseline: 1.9635x; 1.0254x over previous
import jax
import jax.numpy as jnp
from jax.experimental import pallas as pl

_NUM_EMBED = 1024
_EMBED_DIM = 64
_BLK_W = 4096


def _vq_body(e_ref, z_ref, out_ref, sse_ref):
    b = pl.program_id(0)
    w = pl.program_id(1)

    zb = z_ref[0]
    e = e_ref[...]

    eq = jnp.sum(e * e, axis=1, keepdims=True)
    zq = jnp.sum(zb * zb, axis=0, keepdims=True)

    mm2 = jax.lax.dot_general(
        e + e, zb, (((1,), (0,)), ((), ())),
        preferred_element_type=jnp.float32)
    dist = (zq + eq) - mm2

    m = jnp.min(dist, axis=0, keepdims=True)
    iota = jax.lax.broadcasted_iota(jnp.int32, (_NUM_EMBED, _BLK_W), 0)
    sel = jnp.where(dist == m, _NUM_EMBED - iota, 0)
    mx = jnp.max(sel, axis=0, keepdims=True)
    onehot = (sel == mx).astype(jnp.bfloat16)

    e_hi = e.astype(jnp.bfloat16)
    r1 = e - e_hi.astype(jnp.float32)
    e_mid = r1.astype(jnp.bfloat16)
    e_lo = (r1 - e_mid.astype(jnp.float32)).astype(jnp.bfloat16)

    def _gpass(part):
        return jax.lax.dot_general(
            part, onehot, (((0,), (0,)), ((), ())),
            preferred_element_type=jnp.float32)

    q = (_gpass(e_hi) + _gpass(e_mid)) + _gpass(e_lo)

    out_ref[0] = zb + (q - zb)

    d = zb - q
    part = jnp.sum(d * d).reshape(1, 1)

    @pl.when((b == 0) & (w == 0))
    def _init():
        sse_ref[...] = jnp.zeros((1, 1), jnp.float32)

    sse_ref[...] += part


def kernel(z, embed_weight):
    batch, ch, hh, ww = z.shape
    hw = hh * ww
    zr = z.reshape(batch, ch, hw)

    grid = (batch, hw // _BLK_W)
    out, sse = pl.pallas_call(
        _vq_body,
        grid=grid,
        in_specs=[
            pl.BlockSpec((_NUM_EMBED, _EMBED_DIM), lambda b, w: (0, 0)),
            pl.BlockSpec((1, ch, _BLK_W), lambda b, w: (b, 0, w)),
        ],
        out_specs=[
            pl.BlockSpec((1, ch, _BLK_W), lambda b, w: (b, 0, w)),
            pl.BlockSpec((1, 1), lambda b, w: (0, 0)),
        ],
        out_shape=[
            jax.ShapeDtypeStruct((batch, ch, hw), jnp.float32),
            jax.ShapeDtypeStruct((1, 1), jnp.float32),
        ],
    )(embed_weight, zr)

    quantized_st = out.reshape(batch, ch, hh, ww)
    m = sse[0, 0] / z.size
    loss = 0.25 * m + m
    return quantized_st, loss
